# baseline (speedup 1.0000x reference)
import jax
import jax.numpy as jnp
from jax import lax
from jax.experimental import pallas as pl
from jax.experimental.pallas import tpu as pltpu
from jax.experimental.pallas import tpu_sc as plsc

VOCAB = 1000000
DIM = 64
BATCH = 16384
NEG = 20

NC = 2
NS = 16
L = 16
NW = NC * NS
NVREG = DIM // L
SW = DIM
HW = DIM // 2
NPAIR = DIM // 2

CV = 256
NCH1 = VOCAB // CV
REM_OFF = NCH1 * CV

BW = BATCH // NW
CHUNK = 32
NCH2 = BW // CHUNK
NEGC = CHUNK * NEG
IDXW = 128
NGATH = NEGC // IDXW


VB = 256
PB = VB // 2
NBLK = (VOCAB + VB - 1) // VB


def _tc_pack_body(wc_ref, wt_ref, out_ref):
    def pack_tbl(x):
        xb = x.astype(jnp.bfloat16)
        lo = jax.lax.bitcast_convert_type(
            xb[:HW, :], jnp.uint16).astype(jnp.uint32)
        hi = jax.lax.bitcast_convert_type(
            xb[HW:, :], jnp.uint16).astype(jnp.uint32)
        w = (lo | (hi << 16)).astype(jnp.int32).T
        return w.reshape(PB, 2, HW)
    c = pack_tbl(wc_ref[...])
    t = pack_tbl(wt_ref[...])
    out_ref[...] = jnp.concatenate(
        [c[:, 0, :], t[:, 0, :], c[:, 1, :], t[:, 1, :]], axis=1)


@jax.jit
def _tc_pack(wc_t, wt_t):
    return pl.pallas_call(
        _tc_pack_body,
        grid=(NBLK,),
        in_specs=[pl.BlockSpec((DIM, VB), lambda i: (0, i)),
                  pl.BlockSpec((DIM, VB), lambda i: (0, i))],
        out_specs=pl.BlockSpec((PB, 2 * SW), lambda i: (i, 0)),
        out_shape=jax.ShapeDtypeStruct((VOCAB // 2, 2 * SW), jnp.int32),
    )(wc_t, wt_t)


def _score_body(tgt_hbm, ctx_hbm, neg_hbm, s_hbm, pos_hbm, negs_hbm,
                tgt_v, ctx_v, neg_v, t_rows, c_rows, n_rows,
                pos_part, neg_part, pos_out, neg_out, sem):
    wid = lax.axis_index("s") * NC + lax.axis_index("c")
    base = wid * BW
    iota = lax.iota(jnp.int32, L)
    lane = [jnp.full((L,), l, jnp.int32) for l in range(L)]

    def unpack2(ref, r, woff):
        out = []
        for j in range(2):
            w = ref[r, pl.ds(woff + j * L, L)]
            a, b = plsc.unpack(plsc.bitcast(w, jnp.bfloat16),
                               format=plsc.PackFormat.INTERLEAVED)
            out.append(a.astype(jnp.float32))
            out.append(b.astype(jnp.float32))
        return out

    def lane_reduce(part, out, ngroups):
        def g_body(g, c2):
            rows = iota + g * L
            acc = plsc.load_gather(part, [rows, lane[0]])
            for l in range(1, L):
                acc = acc + plsc.load_gather(part, [rows, lane[l]])
            out[pl.ds(g * L, L)] = acc
            return c2
        lax.fori_loop(0, ngroups, g_body, 0)

    def chunk_body(ch, carry):
        off = base + ch * CHUNK

        pltpu.sync_copy(tgt_hbm.at[pl.ds(off, CHUNK)], tgt_v)
        pltpu.sync_copy(ctx_hbm.at[pl.ds(off, CHUNK)], ctx_v)
        pltpu.sync_copy(neg_hbm.at[pl.ds(off * NEG, NEGC)], neg_v)

        cps = [pltpu.async_copy(s_hbm.at[tgt_v], t_rows, sem),
               pltpu.async_copy(s_hbm.at[ctx_v], c_rows, sem)]
        for g in range(NGATH):
            cps.append(pltpu.async_copy(
                s_hbm.at[neg_v.at[pl.ds(g * IDXW, IDXW)]],
                n_rows.at[pl.ds(g * IDXW, IDXW)], sem))
        for cp in cps:
            cp.wait()

        def elem_body(e, c2):
            t = unpack2(t_rows, e, HW)
            c = unpack2(c_rows, e, 0)
            p = t[0] * c[0]
            for j in range(1, NVREG):
                p = p + t[j] * c[j]
            pos_part[e, :] = p
            for k in range(NEG):
                r = e * NEG + k
                n = unpack2(n_rows, r, 0)
                a = t[0] * n[0]
                for j in range(1, NVREG):
                    a = a + t[j] * n[j]
                neg_part[r, :] = a
            return c2

        lax.fori_loop(0, CHUNK, elem_body, 0)
        lane_reduce(pos_part, pos_out, CHUNK // L)
        lane_reduce(neg_part, neg_out, NEGC // L)

        pltpu.sync_copy(pos_out, pos_hbm.at[pl.ds(off, CHUNK)])
        pltpu.sync_copy(neg_out, negs_hbm.at[pl.ds(off * NEG, NEGC)])
        return carry

    lax.fori_loop(0, NCH2, chunk_body, 0)


@jax.jit
def _sc_score(target, context, neg_flat, s_tab):
    mesh = plsc.VectorSubcoreMesh(core_axis_name="c", subcore_axis_name="s")
    return pl.kernel(
        _score_body,
        out_type=(jax.ShapeDtypeStruct((BATCH,), jnp.float32),
                  jax.ShapeDtypeStruct((BATCH * NEG,), jnp.float32)),
        mesh=mesh,
        scratch_types=[
            pltpu.VMEM((CHUNK,), jnp.int32),
            pltpu.VMEM((CHUNK,), jnp.int32),
            pltpu.VMEM((NEGC,), jnp.int32),
            pltpu.VMEM((CHUNK, SW), jnp.int32),
            pltpu.VMEM((CHUNK, SW), jnp.int32),
            pltpu.VMEM((NEGC, SW), jnp.int32),
            pltpu.VMEM((CHUNK, L), jnp.float32),
            pltpu.VMEM((NEGC, L), jnp.float32),
            pltpu.VMEM((CHUNK,), jnp.float32),
            pltpu.VMEM((NEGC,), jnp.float32),
            pltpu.SemaphoreType.DMA,
        ],
        compiler_params=pltpu.CompilerParams(use_tc_tiling_on_sc=False,
                                             needs_layout_passes=False),
    )(target, context, neg_flat, s_tab)


def _tc_loss_body(pos_ref, neg_ref, out_ref):
    pls = jnp.log(jax.nn.sigmoid(pos_ref[...]) + 1e-10)
    nls = jnp.log(jax.nn.sigmoid(-neg_ref[...]) + 1e-10)
    out_ref[0, 0] = -(jnp.sum(pls) + jnp.sum(nls)) / BATCH


@jax.jit
def _tc_loss(pos, neg):
    out = pl.pallas_call(
        _tc_loss_body,
        in_specs=[pl.BlockSpec(memory_space=pltpu.VMEM),
                  pl.BlockSpec(memory_space=pltpu.VMEM)],
        out_specs=pl.BlockSpec(memory_space=pltpu.SMEM),
        out_shape=jax.ShapeDtypeStruct((1, 1), jnp.float32),
    )(pos, neg)
    return out[0, 0]


def kernel(target, context, negatives, W_target, W_context):
    neg_flat = negatives.reshape(BATCH * NEG)
    s_pairs = _tc_pack(W_context.T, W_target.T)
    s_tab = s_pairs.reshape(VOCAB, SW)
    pos, neg = _sc_score(target, context, neg_flat, s_tab)
    return _tc_loss(pos, neg)


# speedup vs baseline: 1.3122x; 1.3122x over previous
import jax
import jax.numpy as jnp
from jax import lax
from jax.experimental import pallas as pl
from jax.experimental.pallas import tpu as pltpu
from jax.experimental.pallas import tpu_sc as plsc

VOCAB = 1000000
DIM = 64
BATCH = 16384
NEG = 20

NC = 2
NS = 16
L = 16
NW = NC * NS
NVREG = DIM // L
SW = DIM
HW = DIM // 2
NPAIR = DIM // 2

CV = 256
NCH1 = VOCAB // CV
REM_OFF = NCH1 * CV

BW = BATCH // NW
CHUNK = 32
NCH2 = BW // CHUNK
NEGC = CHUNK * NEG
IDXW = 128
NGATH = NEGC // IDXW


VB = 512
PB = VB // 2
NBLK = (VOCAB + VB - 1) // VB


def _tc_pack_body(eye_ref, wc_ref, wt_ref, out_ref):
    eye = eye_ref[...]

    def pack_tbl(x):
        xb = x.astype(jnp.bfloat16)
        xt = jax.lax.dot_general(
            eye, xb, (((1,), (1,)), ((), ())),
            preferred_element_type=jnp.float32)
        xtb = xt.astype(jnp.bfloat16)
        lo = jax.lax.bitcast_convert_type(
            xtb[:, :HW], jnp.uint16).astype(jnp.uint32)
        hi = jax.lax.bitcast_convert_type(
            xtb[:, HW:], jnp.uint16).astype(jnp.uint32)
        w = (lo | (hi << 16)).astype(jnp.int32)
        return w.reshape(PB, 2, HW)
    c = pack_tbl(wc_ref[...])
    t = pack_tbl(wt_ref[...])
    out_ref[...] = jnp.concatenate(
        [c[:, 0, :], t[:, 0, :], c[:, 1, :], t[:, 1, :]], axis=1)


@jax.jit
def _tc_pack(wc_t, wt_t):
    eye = jnp.eye(VB, dtype=jnp.bfloat16)
    return pl.pallas_call(
        _tc_pack_body,
        grid=(NBLK,),
        in_specs=[pl.BlockSpec((VB, VB), lambda i: (0, 0)),
                  pl.BlockSpec((DIM, VB), lambda i: (0, i)),
                  pl.BlockSpec((DIM, VB), lambda i: (0, i))],
        out_specs=pl.BlockSpec((PB, 2 * SW), lambda i: (i, 0)),
        out_shape=jax.ShapeDtypeStruct((VOCAB // 2, 2 * SW), jnp.int32),
    )(eye, wc_t, wt_t)


def _score_body(tgt_hbm, ctx_hbm, neg_hbm, s_hbm, pos_hbm, negs_hbm,
                tgt_v, ctx_v, neg_v, t_rows, c_rows, n_rows,
                pos_part, neg_part, pos_out, neg_out, sem):
    wid = lax.axis_index("s") * NC + lax.axis_index("c")
    base = wid * BW
    iota = lax.iota(jnp.int32, L)
    lane = [jnp.full((L,), l, jnp.int32) for l in range(L)]

    def unpack2(ref, r, woff):
        out = []
        for j in range(2):
            w = ref[r, pl.ds(woff + j * L, L)]
            a, b = plsc.unpack(plsc.bitcast(w, jnp.bfloat16),
                               format=plsc.PackFormat.INTERLEAVED)
            out.append(a.astype(jnp.float32))
            out.append(b.astype(jnp.float32))
        return out

    def lane_reduce(part, out, ngroups):
        def g_body(g, c2):
            rows = iota + g * L
            acc = plsc.load_gather(part, [rows, lane[0]])
            for l in range(1, L):
                acc = acc + plsc.load_gather(part, [rows, lane[l]])
            out[pl.ds(g * L, L)] = acc
            return c2
        lax.fori_loop(0, ngroups, g_body, 0)

    def chunk_body(ch, carry):
        off = base + ch * CHUNK

        pltpu.sync_copy(tgt_hbm.at[pl.ds(off, CHUNK)], tgt_v)
        pltpu.sync_copy(ctx_hbm.at[pl.ds(off, CHUNK)], ctx_v)
        pltpu.sync_copy(neg_hbm.at[pl.ds(off * NEG, NEGC)], neg_v)

        cps = [pltpu.async_copy(s_hbm.at[tgt_v], t_rows, sem),
               pltpu.async_copy(s_hbm.at[ctx_v], c_rows, sem)]
        for g in range(NGATH):
            cps.append(pltpu.async_copy(
                s_hbm.at[neg_v.at[pl.ds(g * IDXW, IDXW)]],
                n_rows.at[pl.ds(g * IDXW, IDXW)], sem))
        for cp in cps:
            cp.wait()

        def elem_body(e, c2):
            t = unpack2(t_rows, e, HW)
            c = unpack2(c_rows, e, 0)
            p = t[0] * c[0]
            for j in range(1, NVREG):
                p = p + t[j] * c[j]
            pos_part[e, :] = p
            for k in range(NEG):
                r = e * NEG + k
                n = unpack2(n_rows, r, 0)
                a = t[0] * n[0]
                for j in range(1, NVREG):
                    a = a + t[j] * n[j]
                neg_part[r, :] = a
            return c2

        lax.fori_loop(0, CHUNK, elem_body, 0)
        lane_reduce(pos_part, pos_out, CHUNK // L)
        lane_reduce(neg_part, neg_out, NEGC // L)

        pltpu.sync_copy(pos_out, pos_hbm.at[pl.ds(off, CHUNK)])
        pltpu.sync_copy(neg_out, negs_hbm.at[pl.ds(off * NEG, NEGC)])
        return carry

    lax.fori_loop(0, NCH2, chunk_body, 0)


@jax.jit
def _sc_score(target, context, neg_flat, s_tab):
    mesh = plsc.VectorSubcoreMesh(core_axis_name="c", subcore_axis_name="s")
    return pl.kernel(
        _score_body,
        out_type=(jax.ShapeDtypeStruct((BATCH,), jnp.float32),
                  jax.ShapeDtypeStruct((BATCH * NEG,), jnp.float32)),
        mesh=mesh,
        scratch_types=[
            pltpu.VMEM((CHUNK,), jnp.int32),
            pltpu.VMEM((CHUNK,), jnp.int32),
            pltpu.VMEM((NEGC,), jnp.int32),
            pltpu.VMEM((CHUNK, SW), jnp.int32),
            pltpu.VMEM((CHUNK, SW), jnp.int32),
            pltpu.VMEM((NEGC, SW), jnp.int32),
            pltpu.VMEM((CHUNK, L), jnp.float32),
            pltpu.VMEM((NEGC, L), jnp.float32),
            pltpu.VMEM((CHUNK,), jnp.float32),
            pltpu.VMEM((NEGC,), jnp.float32),
            pltpu.SemaphoreType.DMA,
        ],
        compiler_params=pltpu.CompilerParams(use_tc_tiling_on_sc=False,
                                             needs_layout_passes=False),
    )(target, context, neg_flat, s_tab)


def _tc_loss_body(pos_ref, neg_ref, out_ref):
    pls = jnp.log(jax.nn.sigmoid(pos_ref[...]) + 1e-10)
    nls = jnp.log(jax.nn.sigmoid(-neg_ref[...]) + 1e-10)
    out_ref[0, 0] = -(jnp.sum(pls) + jnp.sum(nls)) / BATCH


@jax.jit
def _tc_loss(pos, neg):
    out = pl.pallas_call(
        _tc_loss_body,
        in_specs=[pl.BlockSpec(memory_space=pltpu.VMEM),
                  pl.BlockSpec(memory_space=pltpu.VMEM)],
        out_specs=pl.BlockSpec(memory_space=pltpu.SMEM),
        out_shape=jax.ShapeDtypeStruct((1, 1), jnp.float32),
    )(pos, neg)
    return out[0, 0]


def kernel(target, context, negatives, W_target, W_context):
    neg_flat = negatives.reshape(BATCH * NEG)
    s_pairs = _tc_pack(W_context.T, W_target.T)
    s_tab = s_pairs.reshape(VOCAB, SW)
    pos, neg = _sc_score(target, context, neg_flat, s_tab)
    return _tc_loss(pos, neg)


# speedup vs baseline: 2.3612x; 1.7995x over previous
import jax
import jax.numpy as jnp
from jax import lax
from jax.experimental import pallas as pl
from jax.experimental.pallas import tpu as pltpu
from jax.experimental.pallas import tpu_sc as plsc

VOCAB = 1000000
DIM = 64
BATCH = 16384
NEG = 20

NC = 2
NS = 16
L = 16
NW = NC * NS
NVREG = DIM // L
SW = DIM
HW = DIM // 2
NPAIR = DIM // 2

CV = 256
NCH1 = VOCAB // CV
REM_OFF = NCH1 * CV

BW = BATCH // NW
CHUNK = 32
NCH2 = BW // CHUNK
NEGC = CHUNK * NEG
IDXW = 128
NGATH = NEGC // IDXW


def _relayout_body(wc_t, wt_t, tail_hbm, s_hbm,
                   src_c0, src_t0, src_c1, src_t1, dst0, dst1,
                   sem0, sem1, semo0, semo1):
    wid = lax.axis_index("s") * NC + lax.axis_index("c")
    base_idx = lax.iota(jnp.int32, L) * SW
    srcs = ((src_c0, src_t0, sem0), (src_c1, src_t1, sem1))
    dsts = ((dst0, semo0), (dst1, semo1))

    def start_in(ch, b):
        off = pl.multiple_of(ch * CV, CV)
        pltpu.async_copy(wc_t.at[:, pl.ds(off, CV)], srcs[b][0], srcs[b][2])
        pltpu.async_copy(wt_t.at[:, pl.ds(off, CV)], srcs[b][1], srcs[b][2])

    def wait_in(b):
        pltpu.make_async_copy(wc_t.at[:, pl.ds(0, CV)],
                              srcs[b][0], srcs[b][2]).wait()
        pltpu.make_async_copy(wt_t.at[:, pl.ds(0, CV)],
                              srcs[b][1], srcs[b][2]).wait()

    def wait_out(b):
        pltpu.make_async_copy(dsts[b][0], s_hbm.at[pl.ds(0, CV * SW)],
                              dsts[b][1]).wait()

    def transpose_chunk(src_c, src_t, dst):
        @plsc.parallel_loop(0, NPAIR)
        def dp_body(dp):
            for toff, src in ((0, src_c), (HW, src_t)):
                col = toff + dp
                for g in range(CV // L):
                    ve = src[2 * dp, pl.ds(g * L, L)]
                    vo = src[2 * dp + 1, pl.ds(g * L, L)]
                    w = plsc.bitcast(
                        plsc.pack(ve, vo, format=plsc.PackFormat.INTERLEAVED),
                        jnp.int32)
                    plsc.store_scatter(
                        dst, [base_idx + (g * L * SW + col)], w)

    nit = NCH1 // (2 * NW) + 2

    @pl.when(wid < NCH1)
    def _():
        start_in(wid, 0)

    def loop_body(it, carry):
        for b in range(2):
            g = 2 * it + b
            ch = wid + g * NW
            nxt = wid + (g + 1) * NW

            @pl.when(jnp.logical_and(g >= 2, (ch - 2 * NW) < NCH1))
            def _():
                wait_out(b)

            @pl.when(ch < NCH1)
            def _():
                wait_in(b)

            @pl.when(nxt < NCH1)
            def _():
                start_in(nxt, 1 - b)

            @pl.when(ch < NCH1)
            def _():
                off = pl.multiple_of(ch * CV, CV)
                transpose_chunk(srcs[b][0], srcs[b][1], dsts[b][0])
                pltpu.async_copy(dsts[b][0],
                                 s_hbm.at[pl.ds(off * SW, CV * SW)],
                                 dsts[b][1])
        return carry

    lax.fori_loop(0, nit, loop_body, 0)

    @pl.when(wid == NW - 1)
    def _():
        pltpu.sync_copy(tail_hbm, dst0.at[pl.ds(0, (VOCAB - REM_OFF) * SW)])
        pltpu.sync_copy(dst0.at[pl.ds(0, (VOCAB - REM_OFF) * SW)],
                        s_hbm.at[pl.ds(REM_OFF * SW,
                                       (VOCAB - REM_OFF) * SW)])


@jax.jit
def _sc_relayout(wc_t, wt_t, tail):
    mesh = plsc.VectorSubcoreMesh(core_axis_name="c", subcore_axis_name="s")
    return pl.kernel(
        _relayout_body,
        out_type=jax.ShapeDtypeStruct((VOCAB * SW,), jnp.int32),
        mesh=mesh,
        scratch_types=[
            pltpu.VMEM((DIM, CV), jnp.float32),
            pltpu.VMEM((DIM, CV), jnp.float32),
            pltpu.VMEM((DIM, CV), jnp.float32),
            pltpu.VMEM((DIM, CV), jnp.float32),
            pltpu.VMEM((CV * SW,), jnp.int32),
            pltpu.VMEM((CV * SW,), jnp.int32),
            pltpu.SemaphoreType.DMA,
            pltpu.SemaphoreType.DMA,
            pltpu.SemaphoreType.DMA,
            pltpu.SemaphoreType.DMA,
        ],
        compiler_params=pltpu.CompilerParams(use_tc_tiling_on_sc=True,
                                             needs_layout_passes=False),
    )(wc_t, wt_t, tail)


def _score_body(tgt_hbm, ctx_hbm, neg_hbm, s_hbm, pos_hbm, negs_hbm,
                tgt_v, ctx_v, neg_v, t_rows, c_rows, n_rows,
                pos_part, neg_part, pos_out, neg_out, sem):
    wid = lax.axis_index("s") * NC + lax.axis_index("c")
    base = wid * BW
    iota = lax.iota(jnp.int32, L)
    lane = [jnp.full((L,), l, jnp.int32) for l in range(L)]

    def unpack2(ref, r, woff):
        out = []
        for j in range(2):
            w = ref[r, pl.ds(woff + j * L, L)]
            a, b = plsc.unpack(plsc.bitcast(w, jnp.bfloat16),
                               format=plsc.PackFormat.INTERLEAVED)
            out.append(a.astype(jnp.float32))
            out.append(b.astype(jnp.float32))
        return out

    def lane_reduce(part, out, ngroups):
        def g_body(g, c2):
            rows = iota + g * L
            acc = plsc.load_gather(part, [rows, lane[0]])
            for l in range(1, L):
                acc = acc + plsc.load_gather(part, [rows, lane[l]])
            out[pl.ds(g * L, L)] = acc
            return c2
        lax.fori_loop(0, ngroups, g_body, 0)

    def chunk_body(ch, carry):
        off = base + ch * CHUNK

        pltpu.sync_copy(tgt_hbm.at[pl.ds(off, CHUNK)], tgt_v)
        pltpu.sync_copy(ctx_hbm.at[pl.ds(off, CHUNK)], ctx_v)
        pltpu.sync_copy(neg_hbm.at[pl.ds(off * NEG, NEGC)], neg_v)

        cps = [pltpu.async_copy(s_hbm.at[tgt_v], t_rows, sem),
               pltpu.async_copy(s_hbm.at[ctx_v], c_rows, sem)]
        for g in range(NGATH):
            cps.append(pltpu.async_copy(
                s_hbm.at[neg_v.at[pl.ds(g * IDXW, IDXW)]],
                n_rows.at[pl.ds(g * IDXW, IDXW)], sem))
        for cp in cps:
            cp.wait()

        def elem_body(e, c2):
            t = unpack2(t_rows, e, HW)
            c = unpack2(c_rows, e, 0)
            p = t[0] * c[0]
            for j in range(1, NVREG):
                p = p + t[j] * c[j]
            pos_part[e, :] = p
            for k in range(NEG):
                r = e * NEG + k
                n = unpack2(n_rows, r, 0)
                a = t[0] * n[0]
                for j in range(1, NVREG):
                    a = a + t[j] * n[j]
                neg_part[r, :] = a
            return c2

        lax.fori_loop(0, CHUNK, elem_body, 0)
        lane_reduce(pos_part, pos_out, CHUNK // L)
        lane_reduce(neg_part, neg_out, NEGC // L)

        pltpu.sync_copy(pos_out, pos_hbm.at[pl.ds(off, CHUNK)])
        pltpu.sync_copy(neg_out, negs_hbm.at[pl.ds(off * NEG, NEGC)])
        return carry

    lax.fori_loop(0, NCH2, chunk_body, 0)


@jax.jit
def _sc_score(target, context, neg_flat, s_tab):
    mesh = plsc.VectorSubcoreMesh(core_axis_name="c", subcore_axis_name="s")
    return pl.kernel(
        _score_body,
        out_type=(jax.ShapeDtypeStruct((BATCH,), jnp.float32),
                  jax.ShapeDtypeStruct((BATCH * NEG,), jnp.float32)),
        mesh=mesh,
        scratch_types=[
            pltpu.VMEM((CHUNK,), jnp.int32),
            pltpu.VMEM((CHUNK,), jnp.int32),
            pltpu.VMEM((NEGC,), jnp.int32),
            pltpu.VMEM((CHUNK, SW), jnp.int32),
            pltpu.VMEM((CHUNK, SW), jnp.int32),
            pltpu.VMEM((NEGC, SW), jnp.int32),
            pltpu.VMEM((CHUNK, L), jnp.float32),
            pltpu.VMEM((NEGC, L), jnp.float32),
            pltpu.VMEM((CHUNK,), jnp.float32),
            pltpu.VMEM((NEGC,), jnp.float32),
            pltpu.SemaphoreType.DMA,
        ],
        compiler_params=pltpu.CompilerParams(use_tc_tiling_on_sc=False,
                                             needs_layout_passes=False),
    )(target, context, neg_flat, s_tab)


def _tc_loss_body(pos_ref, neg_ref, out_ref):
    pls = jnp.log(jax.nn.sigmoid(pos_ref[...]) + 1e-10)
    nls = jnp.log(jax.nn.sigmoid(-neg_ref[...]) + 1e-10)
    out_ref[0, 0] = -(jnp.sum(pls) + jnp.sum(nls)) / BATCH


@jax.jit
def _tc_loss(pos, neg):
    out = pl.pallas_call(
        _tc_loss_body,
        in_specs=[pl.BlockSpec(memory_space=pltpu.VMEM),
                  pl.BlockSpec(memory_space=pltpu.VMEM)],
        out_specs=pl.BlockSpec(memory_space=pltpu.SMEM),
        out_shape=jax.ShapeDtypeStruct((1, 1), jnp.float32),
    )(pos, neg)
    return out[0, 0]


def kernel(target, context, negatives, W_target, W_context):
    neg_flat = negatives.reshape(BATCH * NEG)
    tail_bf = jnp.concatenate(
        [W_context[REM_OFF:], W_target[REM_OFF:]],
        axis=1).astype(jnp.bfloat16)
    tail = jax.lax.bitcast_convert_type(
        tail_bf.reshape(VOCAB - REM_OFF, SW, 2), jnp.int32).reshape(-1)
    s_flat = _sc_relayout(W_context.T, W_target.T, tail)
    s_tab = s_flat.reshape(VOCAB, SW)
    pos, neg = _sc_score(target, context, neg_flat, s_tab)
    return _tc_loss(pos, neg)


# speedup vs baseline: 2.3759x; 1.0062x over previous
import jax
import jax.numpy as jnp
from jax import lax
from jax.experimental import pallas as pl
from jax.experimental.pallas import tpu as pltpu
from jax.experimental.pallas import tpu_sc as plsc

VOCAB = 1000000
DIM = 64
BATCH = 16384
NEG = 20

NC = 2
NS = 16
L = 16
NW = NC * NS
NVREG = DIM // L
SW = DIM
HW = DIM // 2
NPAIR = DIM // 2

CV = 256
NCH1 = VOCAB // CV
REM_OFF = NCH1 * CV

BW = BATCH // NW
CHUNK = 32
NCH2 = BW // CHUNK
NEGC = CHUNK * NEG
IDXW = 128
NGATH = NEGC // IDXW


def _relayout_body(wc_t, wt_t, tail_hbm, s_hbm,
                   src_c0, src_t0, src_c1, src_t1, dst0, dst1,
                   sem0, sem1, semo0, semo1):
    wid = lax.axis_index("s") * NC + lax.axis_index("c")
    base_idx = lax.iota(jnp.int32, L) * SW
    srcs = ((src_c0, src_t0, sem0), (src_c1, src_t1, sem1))
    dsts = ((dst0, semo0), (dst1, semo1))

    def start_in(ch, b):
        off = pl.multiple_of(ch * CV, CV)
        pltpu.async_copy(wc_t.at[:, pl.ds(off, CV)], srcs[b][0], srcs[b][2])
        pltpu.async_copy(wt_t.at[:, pl.ds(off, CV)], srcs[b][1], srcs[b][2])

    def wait_in(b):
        pltpu.make_async_copy(wc_t.at[:, pl.ds(0, CV)],
                              srcs[b][0], srcs[b][2]).wait()
        pltpu.make_async_copy(wt_t.at[:, pl.ds(0, CV)],
                              srcs[b][1], srcs[b][2]).wait()

    def wait_out(b):
        pltpu.make_async_copy(dsts[b][0], s_hbm.at[pl.ds(0, CV * SW)],
                              dsts[b][1]).wait()

    def transpose_chunk(src_c, src_t, dst):
        @plsc.parallel_loop(0, NPAIR, unroll=4)
        def dp_body(dp):
            for toff, src in ((0, src_c), (HW, src_t)):
                col = toff + dp
                for g in range(CV // L):
                    ve = src[2 * dp, pl.ds(g * L, L)]
                    vo = src[2 * dp + 1, pl.ds(g * L, L)]
                    w = plsc.bitcast(
                        plsc.pack(ve, vo, format=plsc.PackFormat.INTERLEAVED),
                        jnp.int32)
                    plsc.store_scatter(
                        dst, [base_idx + (g * L * SW + col)], w)

    nit = NCH1 // (2 * NW) + 2

    @pl.when(wid < NCH1)
    def _():
        start_in(wid, 0)

    def loop_body(it, carry):
        for b in range(2):
            g = 2 * it + b
            ch = wid + g * NW
            nxt = wid + (g + 1) * NW

            @pl.when(jnp.logical_and(g >= 2, (ch - 2 * NW) < NCH1))
            def _():
                wait_out(b)

            @pl.when(ch < NCH1)
            def _():
                wait_in(b)

            @pl.when(nxt < NCH1)
            def _():
                start_in(nxt, 1 - b)

            @pl.when(ch < NCH1)
            def _():
                off = pl.multiple_of(ch * CV, CV)
                transpose_chunk(srcs[b][0], srcs[b][1], dsts[b][0])
                pltpu.async_copy(dsts[b][0],
                                 s_hbm.at[pl.ds(off * SW, CV * SW)],
                                 dsts[b][1])
        return carry

    lax.fori_loop(0, nit, loop_body, 0)

    @pl.when(wid == NW - 1)
    def _():
        pltpu.sync_copy(tail_hbm, dst0.at[pl.ds(0, (VOCAB - REM_OFF) * SW)])
        pltpu.sync_copy(dst0.at[pl.ds(0, (VOCAB - REM_OFF) * SW)],
                        s_hbm.at[pl.ds(REM_OFF * SW,
                                       (VOCAB - REM_OFF) * SW)])


@jax.jit
def _sc_relayout(wc_t, wt_t, tail):
    mesh = plsc.VectorSubcoreMesh(core_axis_name="c", subcore_axis_name="s")
    return pl.kernel(
        _relayout_body,
        out_type=jax.ShapeDtypeStruct((VOCAB * SW,), jnp.int32),
        mesh=mesh,
        scratch_types=[
            pltpu.VMEM((DIM, CV), jnp.float32),
            pltpu.VMEM((DIM, CV), jnp.float32),
            pltpu.VMEM((DIM, CV), jnp.float32),
            pltpu.VMEM((DIM, CV), jnp.float32),
            pltpu.VMEM((CV * SW,), jnp.int32),
            pltpu.VMEM((CV * SW,), jnp.int32),
            pltpu.SemaphoreType.DMA,
            pltpu.SemaphoreType.DMA,
            pltpu.SemaphoreType.DMA,
            pltpu.SemaphoreType.DMA,
        ],
        compiler_params=pltpu.CompilerParams(use_tc_tiling_on_sc=True,
                                             needs_layout_passes=False),
    )(wc_t, wt_t, tail)


def _score_body(tgt_hbm, ctx_hbm, neg_hbm, s_hbm, pos_hbm, negs_hbm,
                tgt_v, ctx_v, neg_v, t_rows, c_rows, n_rows,
                pos_part, neg_part, pos_out, neg_out, sem):
    wid = lax.axis_index("s") * NC + lax.axis_index("c")
    base = wid * BW
    iota = lax.iota(jnp.int32, L)
    lane = [jnp.full((L,), l, jnp.int32) for l in range(L)]

    def unpack2(ref, r, woff):
        out = []
        for j in range(2):
            w = ref[r, pl.ds(woff + j * L, L)]
            a, b = plsc.unpack(plsc.bitcast(w, jnp.bfloat16),
                               format=plsc.PackFormat.INTERLEAVED)
            out.append(a.astype(jnp.float32))
            out.append(b.astype(jnp.float32))
        return out

    def lane_reduce(part, out, ngroups):
        def g_body(g, c2):
            rows = iota + g * L
            acc = plsc.load_gather(part, [rows, lane[0]])
            for l in range(1, L):
                acc = acc + plsc.load_gather(part, [rows, lane[l]])
            out[pl.ds(g * L, L)] = acc
            return c2
        lax.fori_loop(0, ngroups, g_body, 0)

    def chunk_body(ch, carry):
        off = base + ch * CHUNK

        pltpu.sync_copy(tgt_hbm.at[pl.ds(off, CHUNK)], tgt_v)
        pltpu.sync_copy(ctx_hbm.at[pl.ds(off, CHUNK)], ctx_v)
        pltpu.sync_copy(neg_hbm.at[pl.ds(off * NEG, NEGC)], neg_v)

        cps = [pltpu.async_copy(s_hbm.at[tgt_v], t_rows, sem),
               pltpu.async_copy(s_hbm.at[ctx_v], c_rows, sem)]
        for g in range(NGATH):
            cps.append(pltpu.async_copy(
                s_hbm.at[neg_v.at[pl.ds(g * IDXW, IDXW)]],
                n_rows.at[pl.ds(g * IDXW, IDXW)], sem))
        for cp in cps:
            cp.wait()

        def elem_body(e, c2):
            t = unpack2(t_rows, e, HW)
            c = unpack2(c_rows, e, 0)
            p = t[0] * c[0]
            for j in range(1, NVREG):
                p = p + t[j] * c[j]
            pos_part[e, :] = p
            for k in range(NEG):
                r = e * NEG + k
                n = unpack2(n_rows, r, 0)
                a = t[0] * n[0]
                for j in range(1, NVREG):
                    a = a + t[j] * n[j]
                neg_part[r, :] = a
            return c2

        lax.fori_loop(0, CHUNK, elem_body, 0)
        lane_reduce(pos_part, pos_out, CHUNK // L)
        lane_reduce(neg_part, neg_out, NEGC // L)

        pltpu.sync_copy(pos_out, pos_hbm.at[pl.ds(off, CHUNK)])
        pltpu.sync_copy(neg_out, negs_hbm.at[pl.ds(off * NEG, NEGC)])
        return carry

    lax.fori_loop(0, NCH2, chunk_body, 0)


@jax.jit
def _sc_score(target, context, neg_flat, s_tab):
    mesh = plsc.VectorSubcoreMesh(core_axis_name="c", subcore_axis_name="s")
    return pl.kernel(
        _score_body,
        out_type=(jax.ShapeDtypeStruct((BATCH,), jnp.float32),
                  jax.ShapeDtypeStruct((BATCH * NEG,), jnp.float32)),
        mesh=mesh,
        scratch_types=[
            pltpu.VMEM((CHUNK,), jnp.int32),
            pltpu.VMEM((CHUNK,), jnp.int32),
            pltpu.VMEM((NEGC,), jnp.int32),
            pltpu.VMEM((CHUNK, SW), jnp.int32),
            pltpu.VMEM((CHUNK, SW), jnp.int32),
            pltpu.VMEM((NEGC, SW), jnp.int32),
            pltpu.VMEM((CHUNK, L), jnp.float32),
            pltpu.VMEM((NEGC, L), jnp.float32),
            pltpu.VMEM((CHUNK,), jnp.float32),
            pltpu.VMEM((NEGC,), jnp.float32),
            pltpu.SemaphoreType.DMA,
        ],
        compiler_params=pltpu.CompilerParams(use_tc_tiling_on_sc=False,
                                             needs_layout_passes=False),
    )(target, context, neg_flat, s_tab)


def _tc_loss_body(pos_ref, neg_ref, out_ref):
    pls = jnp.log(jax.nn.sigmoid(pos_ref[...]) + 1e-10)
    nls = jnp.log(jax.nn.sigmoid(-neg_ref[...]) + 1e-10)
    out_ref[0, 0] = -(jnp.sum(pls) + jnp.sum(nls)) / BATCH


@jax.jit
def _tc_loss(pos, neg):
    out = pl.pallas_call(
        _tc_loss_body,
        in_specs=[pl.BlockSpec(memory_space=pltpu.VMEM),
                  pl.BlockSpec(memory_space=pltpu.VMEM)],
        out_specs=pl.BlockSpec(memory_space=pltpu.SMEM),
        out_shape=jax.ShapeDtypeStruct((1, 1), jnp.float32),
    )(pos, neg)
    return out[0, 0]


def kernel(target, context, negatives, W_target, W_context):
    neg_flat = negatives.reshape(BATCH * NEG)
    tail_bf = jnp.concatenate(
        [W_context[REM_OFF:], W_target[REM_OFF:]],
        axis=1).astype(jnp.bfloat16)
    tail = jax.lax.bitcast_convert_type(
        tail_bf.reshape(VOCAB - REM_OFF, SW, 2), jnp.int32).reshape(-1)
    s_flat = _sc_relayout(W_context.T, W_target.T, tail)
    s_tab = s_flat.reshape(VOCAB, SW)
    pos, neg = _sc_score(target, context, neg_flat, s_tab)
    return _tc_loss(pos, neg)


# speedup vs baseline: 2.4316x; 1.0234x over previous
import jax
import jax.numpy as jnp
from jax import lax
from jax.experimental import pallas as pl
from jax.experimental.pallas import tpu as pltpu
from jax.experimental.pallas import tpu_sc as plsc

VOCAB = 1000000
DIM = 64
BATCH = 16384
NEG = 20

NC = 2
NS = 16
L = 16
NW = NC * NS
NVREG = DIM // L
SW = DIM
HW = DIM // 2
NPAIR = DIM // 2

CV = 256
NCH1 = VOCAB // CV
REM_OFF = NCH1 * CV

BW = BATCH // NW
CHUNK = 64
NCH2 = BW // CHUNK
NEGC = CHUNK * NEG
IDXW = 128
NGATH = NEGC // IDXW


def _relayout_body(wc_t, wt_t, tail_hbm, s_hbm,
                   src_c0, src_t0, src_c1, src_t1, dst0, dst1,
                   sem0, sem1, semo0, semo1):
    wid = lax.axis_index("s") * NC + lax.axis_index("c")
    base_idx = lax.iota(jnp.int32, L) * SW
    srcs = ((src_c0, src_t0, sem0), (src_c1, src_t1, sem1))
    dsts = ((dst0, semo0), (dst1, semo1))

    def start_in(ch, b):
        off = pl.multiple_of(ch * CV, CV)
        pltpu.async_copy(wc_t.at[:, pl.ds(off, CV)], srcs[b][0], srcs[b][2])
        pltpu.async_copy(wt_t.at[:, pl.ds(off, CV)], srcs[b][1], srcs[b][2])

    def wait_in(b):
        pltpu.make_async_copy(wc_t.at[:, pl.ds(0, CV)],
                              srcs[b][0], srcs[b][2]).wait()
        pltpu.make_async_copy(wt_t.at[:, pl.ds(0, CV)],
                              srcs[b][1], srcs[b][2]).wait()

    def wait_out(b):
        pltpu.make_async_copy(dsts[b][0], s_hbm.at[pl.ds(0, CV * SW)],
                              dsts[b][1]).wait()

    def transpose_chunk(src_c, src_t, dst):
        @plsc.parallel_loop(0, NPAIR, unroll=4)
        def dp_body(dp):
            for toff, src in ((0, src_c), (HW, src_t)):
                col = toff + dp
                for g in range(CV // L):
                    ve = src[2 * dp, pl.ds(g * L, L)]
                    vo = src[2 * dp + 1, pl.ds(g * L, L)]
                    w = plsc.bitcast(
                        plsc.pack(ve, vo, format=plsc.PackFormat.INTERLEAVED),
                        jnp.int32)
                    plsc.store_scatter(
                        dst, [base_idx + (g * L * SW + col)], w)

    nit = NCH1 // (2 * NW) + 2

    @pl.when(wid < NCH1)
    def _():
        start_in(wid, 0)

    def loop_body(it, carry):
        for b in range(2):
            g = 2 * it + b
            ch = wid + g * NW
            nxt = wid + (g + 1) * NW

            @pl.when(jnp.logical_and(g >= 2, (ch - 2 * NW) < NCH1))
            def _():
                wait_out(b)

            @pl.when(ch < NCH1)
            def _():
                wait_in(b)

            @pl.when(nxt < NCH1)
            def _():
                start_in(nxt, 1 - b)

            @pl.when(ch < NCH1)
            def _():
                off = pl.multiple_of(ch * CV, CV)
                transpose_chunk(srcs[b][0], srcs[b][1], dsts[b][0])
                pltpu.async_copy(dsts[b][0],
                                 s_hbm.at[pl.ds(off * SW, CV * SW)],
                                 dsts[b][1])
        return carry

    lax.fori_loop(0, nit, loop_body, 0)

    @pl.when(wid == NW - 1)
    def _():
        pltpu.sync_copy(tail_hbm, dst0.at[pl.ds(0, (VOCAB - REM_OFF) * SW)])
        pltpu.sync_copy(dst0.at[pl.ds(0, (VOCAB - REM_OFF) * SW)],
                        s_hbm.at[pl.ds(REM_OFF * SW,
                                       (VOCAB - REM_OFF) * SW)])


@jax.jit
def _sc_relayout(wc_t, wt_t, tail):
    mesh = plsc.VectorSubcoreMesh(core_axis_name="c", subcore_axis_name="s")
    return pl.kernel(
        _relayout_body,
        out_type=jax.ShapeDtypeStruct((VOCAB * SW,), jnp.int32),
        mesh=mesh,
        scratch_types=[
            pltpu.VMEM((DIM, CV), jnp.float32),
            pltpu.VMEM((DIM, CV), jnp.float32),
            pltpu.VMEM((DIM, CV), jnp.float32),
            pltpu.VMEM((DIM, CV), jnp.float32),
            pltpu.VMEM((CV * SW,), jnp.int32),
            pltpu.VMEM((CV * SW,), jnp.int32),
            pltpu.SemaphoreType.DMA,
            pltpu.SemaphoreType.DMA,
            pltpu.SemaphoreType.DMA,
            pltpu.SemaphoreType.DMA,
        ],
        compiler_params=pltpu.CompilerParams(use_tc_tiling_on_sc=True,
                                             needs_layout_passes=False),
    )(wc_t, wt_t, tail)


def _score_body(tgt_hbm, ctx_hbm, neg_hbm, s_hbm, pos_hbm, negs_hbm,
                tgt_v, ctx_v, neg_v, t_rows, c_rows, n_rows,
                pos_part, neg_part, pos_out, neg_out, sem):
    wid = lax.axis_index("s") * NC + lax.axis_index("c")
    base = wid * BW
    iota = lax.iota(jnp.int32, L)
    lane = [jnp.full((L,), l, jnp.int32) for l in range(L)]

    pltpu.sync_copy(tgt_hbm.at[pl.ds(base, BW)], tgt_v)
    pltpu.sync_copy(ctx_hbm.at[pl.ds(base, BW)], ctx_v)
    pltpu.sync_copy(neg_hbm.at[pl.ds(base * NEG, BW * NEG)], neg_v)

    def unpack2(ref, r, woff):
        out = []
        for j in range(2):
            w = ref[r, pl.ds(woff + j * L, L)]
            a, b = plsc.unpack(plsc.bitcast(w, jnp.bfloat16),
                               format=plsc.PackFormat.INTERLEAVED)
            out.append(a.astype(jnp.float32))
            out.append(b.astype(jnp.float32))
        return out

    def lane_reduce(part, out, ngroups):
        def g_body(g, c2):
            rows = iota + g * L
            acc = plsc.load_gather(part, [rows, lane[0]])
            for l in range(1, L):
                acc = acc + plsc.load_gather(part, [rows, lane[l]])
            out[pl.ds(g * L, L)] = acc
            return c2
        lax.fori_loop(0, ngroups, g_body, 0)

    def chunk_body(ch, carry):
        off = base + ch * CHUNK
        ioff = ch * CHUNK

        cps = [pltpu.async_copy(
                   s_hbm.at[tgt_v.at[pl.ds(ioff, CHUNK)]], t_rows, sem),
               pltpu.async_copy(
                   s_hbm.at[ctx_v.at[pl.ds(ioff, CHUNK)]], c_rows, sem)]
        for g in range(NGATH):
            cps.append(pltpu.async_copy(
                s_hbm.at[neg_v.at[pl.ds(ioff * NEG + g * IDXW, IDXW)]],
                n_rows.at[pl.ds(g * IDXW, IDXW)], sem))
        for cp in cps:
            cp.wait()

        def elem_body(e, c2):
            t = unpack2(t_rows, e, HW)
            c = unpack2(c_rows, e, 0)
            p = t[0] * c[0]
            for j in range(1, NVREG):
                p = p + t[j] * c[j]
            pos_part[e, :] = p
            for k in range(NEG):
                r = e * NEG + k
                n = unpack2(n_rows, r, 0)
                a = t[0] * n[0]
                for j in range(1, NVREG):
                    a = a + t[j] * n[j]
                neg_part[r, :] = a
            return c2

        lax.fori_loop(0, CHUNK, elem_body, 0)
        lane_reduce(pos_part, pos_out, CHUNK // L)
        lane_reduce(neg_part, neg_out, NEGC // L)

        pltpu.sync_copy(pos_out, pos_hbm.at[pl.ds(off, CHUNK)])
        pltpu.sync_copy(neg_out, negs_hbm.at[pl.ds(off * NEG, NEGC)])
        return carry

    lax.fori_loop(0, NCH2, chunk_body, 0)


@jax.jit
def _sc_score(target, context, neg_flat, s_tab):
    mesh = plsc.VectorSubcoreMesh(core_axis_name="c", subcore_axis_name="s")
    return pl.kernel(
        _score_body,
        out_type=(jax.ShapeDtypeStruct((BATCH,), jnp.float32),
                  jax.ShapeDtypeStruct((BATCH * NEG,), jnp.float32)),
        mesh=mesh,
        scratch_types=[
            pltpu.VMEM((BW,), jnp.int32),
            pltpu.VMEM((BW,), jnp.int32),
            pltpu.VMEM((BW * NEG,), jnp.int32),
            pltpu.VMEM((CHUNK, SW), jnp.int32),
            pltpu.VMEM((CHUNK, SW), jnp.int32),
            pltpu.VMEM((NEGC, SW), jnp.int32),
            pltpu.VMEM((CHUNK, L), jnp.float32),
            pltpu.VMEM((NEGC, L), jnp.float32),
            pltpu.VMEM((CHUNK,), jnp.float32),
            pltpu.VMEM((NEGC,), jnp.float32),
            pltpu.SemaphoreType.DMA,
        ],
        compiler_params=pltpu.CompilerParams(use_tc_tiling_on_sc=False,
                                             needs_layout_passes=False),
    )(target, context, neg_flat, s_tab)


def _tc_loss_body(pos_ref, neg_ref, out_ref):
    pls = jnp.log(jax.nn.sigmoid(pos_ref[...]) + 1e-10)
    nls = jnp.log(jax.nn.sigmoid(-neg_ref[...]) + 1e-10)
    out_ref[0, 0] = -(jnp.sum(pls) + jnp.sum(nls)) / BATCH


@jax.jit
def _tc_loss(pos, neg):
    out = pl.pallas_call(
        _tc_loss_body,
        in_specs=[pl.BlockSpec(memory_space=pltpu.VMEM),
                  pl.BlockSpec(memory_space=pltpu.VMEM)],
        out_specs=pl.BlockSpec(memory_space=pltpu.SMEM),
        out_shape=jax.ShapeDtypeStruct((1, 1), jnp.float32),
    )(pos, neg)
    return out[0, 0]


def kernel(target, context, negatives, W_target, W_context):
    neg_flat = negatives.reshape(BATCH * NEG)
    tail_bf = jnp.concatenate(
        [W_context[REM_OFF:], W_target[REM_OFF:]],
        axis=1).astype(jnp.bfloat16)
    tail = jax.lax.bitcast_convert_type(
        tail_bf.reshape(VOCAB - REM_OFF, SW, 2), jnp.int32).reshape(-1)
    s_flat = _sc_relayout(W_context.T, W_target.T, tail)
    s_tab = s_flat.reshape(VOCAB, SW)
    pos, neg = _sc_score(target, context, neg_flat, s_tab)
    return _tc_loss(pos, neg)
